# Initial kernel scaffold; baseline (speedup 1.0000x reference)
import jax
import jax.numpy as jnp
from jax.experimental import pallas as pl


def kernel(inputs, table, pos_table):
    raise NotImplementedError("write your pallas kernel here")


# speedup vs baseline: 3.0789x; 3.0789x over previous
import functools

import jax
import jax.numpy as jnp
from jax import lax
from jax.experimental import pallas as pl
from jax.experimental.pallas import tpu as pltpu
from jax.experimental.pallas import tpu_sc as plsc

BATCH = 4096
SEQ = 200
DIM = 64
LANES = 16
NW = 32
ROWS_PER_W = BATCH * SEQ // NW
RB = 2
STEP_ROWS = RB * SEQ
STEPS = ROWS_PER_W // STEP_ROWS
CHUNK = 100
NCHUNK = STEP_ROWS // CHUNK


def _sc_body(idx_hbm, table_hbm, pos_hbm, out_hbm, idx_v, rows_v, pos_v, sem):
    cid = lax.axis_index("c")
    sid = lax.axis_index("s")
    wid = sid * 2 + cid

    pltpu.sync_copy(pos_hbm, pos_v)

    def step(t, carry):
        row0 = wid * STEPS * NCHUNK + t * NCHUNK
        pltpu.sync_copy(idx_hbm.at[pl.ds(row0, NCHUNK)], idx_v)
        for i in range(NCHUNK):
            pltpu.async_copy(
                table_hbm.at[idx_v.at[i]],
                rows_v.at[pl.ds(i * CHUNK, CHUNK)],
                sem,
            ).wait()
        def addrow(r, c2):
            for k in range(DIM // LANES):
                p = pos_v[r, pl.ds(k * LANES, LANES)]
                for part in range(RB):
                    rr = part * SEQ + r
                    rows_v[rr, pl.ds(k * LANES, LANES)] = (
                        rows_v[rr, pl.ds(k * LANES, LANES)] + p
                    )
            return c2
        lax.fori_loop(0, SEQ, addrow, 0)
        pltpu.sync_copy(
            rows_v,
            out_hbm.at[pl.ds(wid * ROWS_PER_W + t * STEP_ROWS, STEP_ROWS)],
        )
        return carry

    lax.fori_loop(0, STEPS, step, 0)


_sc_embed = functools.partial(
    pl.kernel,
    out_type=jax.ShapeDtypeStruct((BATCH * SEQ, DIM), jnp.float32),
    mesh=plsc.VectorSubcoreMesh(
        core_axis_name="c", subcore_axis_name="s", num_cores=2, num_subcores=16
    ),
    scratch_types=[
        pltpu.VMEM((NCHUNK, CHUNK), jnp.int32),
        pltpu.VMEM((STEP_ROWS, DIM), jnp.float32),
        pltpu.VMEM((SEQ, DIM), jnp.float32),
        pltpu.SemaphoreType.DMA,
    ],
    compiler_params=pltpu.CompilerParams(use_tc_tiling_on_sc=False),
)(_sc_body)


def kernel(inputs, table, pos_table):
    idx = inputs.reshape(-1, CHUNK)
    out = _sc_embed(idx, table, pos_table)
    return out.reshape(BATCH, SEQ, DIM)


# speedup vs baseline: 3.7246x; 1.2097x over previous
import functools

import jax
import jax.numpy as jnp
from jax import lax
from jax.experimental import pallas as pl
from jax.experimental.pallas import tpu as pltpu
from jax.experimental.pallas import tpu_sc as plsc

BATCH = 4096
SEQ = 200
DIM = 64
LANES = 16
NW = 32
ROWS_PER_W = BATCH * SEQ // NW
RB = 2
STEP_ROWS = RB * SEQ
STEPS = ROWS_PER_W // STEP_ROWS
CHUNK = 100
NCHUNK = STEP_ROWS // CHUNK


def _sc_body(idx_hbm, table_hbm, pos_hbm, out_hbm,
             idx_v, rows_v, pos_v, sem_g0, sem_g1, sem_o0, sem_o1):
    cid = lax.axis_index("c")
    sid = lax.axis_index("s")
    wid = sid * 2 + cid
    sem_g = (sem_g0, sem_g1)
    sem_o = (sem_o0, sem_o1)

    pltpu.sync_copy(pos_hbm, pos_v)
    pltpu.sync_copy(idx_hbm.at[pl.ds(wid * STEPS * NCHUNK, STEPS * NCHUNK)],
                    idx_v)

    def fire_gathers(t, p):
        for i in range(NCHUNK):
            pltpu.async_copy(
                table_hbm.at[idx_v.at[t * NCHUNK + i]],
                rows_v.at[p].at[pl.ds(i * CHUNK, CHUNK)],
                sem_g[p],
            )

    def drain_gathers(p):
        pltpu.make_async_copy(
            table_hbm.at[pl.ds(0, STEP_ROWS)], rows_v.at[p], sem_g[p]
        ).wait()

    def drain_store(p):
        pltpu.make_async_copy(
            rows_v.at[p], out_hbm.at[pl.ds(0, STEP_ROWS)], sem_o[p]
        ).wait()

    def add_pos(p):
        rows = rows_v.at[p]

        @pl.loop(0, SEQ, unroll=4)
        def _(r):
            for k in range(DIM // LANES):
                pv = pos_v[r, pl.ds(k * LANES, LANES)]
                for part in range(RB):
                    rr = part * SEQ + r
                    rows[rr, pl.ds(k * LANES, LANES)] = (
                        rows[rr, pl.ds(k * LANES, LANES)] + pv
                    )

    fire_gathers(0, 0)

    @pl.loop(0, STEPS, step=2)
    def _(tt):
        for p in range(2):
            t = tt + p
            q = 1 - p

            @pl.when(t + 1 < STEPS)
            def _prefetch():
                @pl.when(t >= 1)
                def _reclaim():
                    drain_store(q)
                fire_gathers(t + 1, q)

            drain_gathers(p)
            add_pos(p)
            pltpu.async_copy(
                rows_v.at[p],
                out_hbm.at[pl.ds(wid * ROWS_PER_W + t * STEP_ROWS, STEP_ROWS)],
                sem_o[p],
            )

    drain_store(0)
    drain_store(1)


_sc_embed = functools.partial(
    pl.kernel,
    out_type=jax.ShapeDtypeStruct((BATCH * SEQ, DIM), jnp.float32),
    mesh=plsc.VectorSubcoreMesh(
        core_axis_name="c", subcore_axis_name="s", num_cores=2, num_subcores=16
    ),
    scratch_types=[
        pltpu.VMEM((STEPS * NCHUNK, CHUNK), jnp.int32),
        pltpu.VMEM((2, STEP_ROWS, DIM), jnp.float32),
        pltpu.VMEM((SEQ, DIM), jnp.float32),
        pltpu.SemaphoreType.DMA,
        pltpu.SemaphoreType.DMA,
        pltpu.SemaphoreType.DMA,
        pltpu.SemaphoreType.DMA,
    ],
    compiler_params=pltpu.CompilerParams(use_tc_tiling_on_sc=False),
)(_sc_body)


def kernel(inputs, table, pos_table):
    idx = inputs.reshape(-1, CHUNK)
    out = _sc_embed(idx, table, pos_table)
    return out.reshape(BATCH, SEQ, DIM)
